# baseline (speedup 1.0000x reference)
import jax
import jax.numpy as jnp
from jax.experimental import pallas as pl
from jax.experimental.pallas import tpu as pltpu

_ROWS_PER_BLOCK = 2048


def _rowmax_block(x_ref, o_ref):
    x = x_ref[...]
    r, k = x.shape
    if k % 128 == 0 and k > 128:
        folded = jnp.max(x.reshape(r, k // 128, 128), axis=1)
    else:
        folded = x
    o_ref[...] = jnp.max(folded, axis=-1, keepdims=True)


def kernel(x):
    n, c, h, w = x.shape
    rows, cols = n * c, h * w
    x2d = x.reshape(rows, cols)
    itemsize = jnp.dtype(x.dtype).itemsize

    br = min(_ROWS_PER_BLOCK, rows)
    grid = pl.cdiv(rows, br)

    out = pl.pallas_call(
        _rowmax_block,
        out_shape=jax.ShapeDtypeStruct((rows, 1), x.dtype),
        grid=(grid,),
        in_specs=[pl.BlockSpec((br, cols), lambda i: (i, 0))],
        out_specs=pl.BlockSpec((br, 1), lambda i: (i, 0)),
        compiler_params=pltpu.CompilerParams(
            dimension_semantics=("parallel",),
            vmem_limit_bytes=min(4 * br * cols * itemsize, 48 << 20),
        ),
        cost_estimate=pl.CostEstimate(
            flops=rows * cols,
            transcendentals=0,
            bytes_accessed=rows * cols * itemsize + rows * itemsize,
        ),
    )(x2d)

    pooled = out.reshape(n, c)
    squeezed = tuple(d for d in (n, c) if d != 1)
    y = pooled.reshape(squeezed)
    if n == 1:
        y = y[None, ...]
    return y


# speedup vs baseline: 1.0379x; 1.0379x over previous
import jax
import jax.numpy as jnp
from jax.experimental import pallas as pl
from jax.experimental.pallas import tpu as pltpu

_ROWS_PER_BLOCK = 2048


def _rowmax_block(x_ref, o_ref):
    o_ref[...] = jnp.max(x_ref[...], axis=-1, keepdims=True)


def kernel(x):
    n, c, h, w = x.shape
    rows, cols = n * c, h * w
    x2d = x.reshape(rows, cols)
    itemsize = jnp.dtype(x.dtype).itemsize

    br = min(_ROWS_PER_BLOCK, rows)
    grid = pl.cdiv(rows, br)

    out = pl.pallas_call(
        _rowmax_block,
        out_shape=jax.ShapeDtypeStruct((rows, 1), x.dtype),
        grid=(grid,),
        in_specs=[pl.BlockSpec((br, cols), lambda i: (i, 0))],
        out_specs=pl.BlockSpec((br, 1), lambda i: (i, 0)),
        compiler_params=pltpu.CompilerParams(
            dimension_semantics=("parallel",),
            vmem_limit_bytes=min(4 * br * cols * itemsize, 48 << 20),
        ),
        cost_estimate=pl.CostEstimate(
            flops=rows * cols,
            transcendentals=0,
            bytes_accessed=rows * cols * itemsize + rows * itemsize,
        ),
    )(x2d)

    pooled = out.reshape(n, c)
    squeezed = tuple(d for d in (n, c) if d != 1)
    y = pooled.reshape(squeezed)
    if n == 1:
        y = y[None, ...]
    return y
